# baseline (speedup 1.0000x reference)
import jax
import jax.numpy as jnp
from jax.experimental import pallas as pl
from jax.experimental.pallas import tpu as pltpu

SM = 1e-5
B, C = 2, 4
DHW = 96 * 96 * 96
ROWS = DHW // 128
N = B * C * DHW


def _tc_body(x_ref, l_ref, out_ref, acc_ref):
    i = pl.program_id(0)
    c = i % C
    x = x_ref[0]
    lbl = l_ref[0]
    t = (lbl == c + 1).astype(jnp.float32)
    u = jnp.exp(-jnp.abs(x))
    inv = 1.0 / (1.0 + u)
    sig = jnp.where(x >= 0, inv, u * inv)
    s1 = jnp.sum(sig)
    g1 = jnp.sum(sig * t)
    h = jnp.sum(t)
    bce = jnp.sum(jnp.maximum(x, 0.0) - x * t + jnp.log1p(u))
    dc = (2.0 * g1 + SM) / (s1 + h + SM)

    @pl.when(i == 0)
    def _init():
        acc_ref[0] = 0.0
        acc_ref[1] = 0.0

    acc_ref[0] += dc
    acc_ref[1] += bce

    @pl.when(i == B * C - 1)
    def _fin():
        out_ref[0] = (1.0 - acc_ref[0] / (B * C)) + acc_ref[1] / N


def kernel(net_output, target):
    x = net_output.reshape(B * C, ROWS, 128)
    lbl = target.astype(jnp.int32).reshape(B, ROWS, 128)
    out = pl.pallas_call(
        _tc_body,
        grid=(B * C,),
        in_specs=[
            pl.BlockSpec((1, ROWS, 128), lambda i: (i, 0, 0)),
            pl.BlockSpec((1, ROWS, 128), lambda i: (i // C, 0, 0)),
        ],
        out_specs=pl.BlockSpec(memory_space=pltpu.SMEM),
        out_shape=jax.ShapeDtypeStruct((1,), jnp.float32),
        scratch_shapes=[pltpu.SMEM((2,), jnp.float32)],
    )(x, lbl)
    return out[0]
